# baseline (speedup 1.0000x reference)
import jax
import jax.numpy as jnp
from jax.experimental import pallas as pl
from jax.experimental.pallas import tpu as pltpu

B = 32768
INPUT_DIM = 768
HIDDEN = 768
USER_DIM = 128
E = 8
TOPK = 2

BLOCK_B = 2048


def _leaky_relu(v):
    return jnp.where(v >= 0, v, 0.2 * v)


def _dot(a, b):
    return jnp.dot(a, b, preferred_element_type=jnp.float32)


def _dot_t(a, b):
    return jax.lax.dot_general(
        a, b, ((( 1,), (1,)), ((), ())),
        preferred_element_type=jnp.float32)


def _fused_kernel(x_ref, u_ref, w1_ref, b1_ref, w2_ref, b2_ref,
                  gw1_ref, gb1_ref, gw2t_ref, gb2t_ref,
                  woutt_ref, boutt_ref,
                  total_ref, gate_ref):
    h = _leaky_relu(
        _dot(x_ref[...].astype(jnp.bfloat16), w1_ref[...]) + b1_ref[...])
    features = _leaky_relu(
        _dot(h.astype(jnp.bfloat16), w2_ref[...]) + b2_ref[...])

    g = jnp.maximum(_dot(u_ref[...], gw1_ref[...]) + gb1_ref[...], 0.0)
    logits = _dot_t(gw2t_ref[...], g) + gb2t_ref[...]

    m = jnp.max(logits, axis=0, keepdims=True)
    ex = jnp.exp(logits - m)
    gw = ex / jnp.sum(ex, axis=0, keepdims=True)

    lane = jax.lax.broadcasted_iota(jnp.int32, gw.shape, 0)
    m1 = jnp.max(gw, axis=0, keepdims=True)
    i1 = jnp.min(jnp.where(gw == m1, lane, E), axis=0, keepdims=True)
    sel1 = lane == i1
    gw_rest = jnp.where(sel1, -jnp.inf, gw)
    m2 = jnp.max(gw_rest, axis=0, keepdims=True)
    i2 = jnp.min(jnp.where(gw_rest == m2, lane, E), axis=0, keepdims=True)
    gw_masked = jnp.where(sel1 | (lane == i2), gw, 0.0)
    gw_final = gw_masked / (jnp.sum(gw_masked, axis=0, keepdims=True) + 1e-9)

    out9 = _dot_t(woutt_ref[...], features) + boutt_ref[...]
    total = jnp.sum(out9[:E] * gw_final, axis=0, keepdims=True) + out9[E:E + 1]

    total_ref[...] = total
    gate_ref[...] = gw_final


@jax.jit
def kernel(x, user_emb, enc_w1, enc_b1, enc_w2, enc_b2,
           gate_w1, gate_b1, gate_w2, gate_b2,
           expert_w, expert_b, shared_w, shared_b):
    w_out_t = jnp.concatenate([expert_w[:, :, 0], shared_w.T], axis=0)
    b_out_t = jnp.concatenate([expert_b[:, 0], shared_b])[:, None]

    grid = (B // BLOCK_B,)
    full = lambda i: (0, 0)
    row = lambda i: (i, 0)
    col = lambda i: (0, i)

    total_t, gate_t = pl.pallas_call(
        _fused_kernel,
        grid=grid,
        in_specs=[
            pl.BlockSpec((BLOCK_B, INPUT_DIM), row),
            pl.BlockSpec((BLOCK_B, USER_DIM), row),
            pl.BlockSpec((INPUT_DIM, HIDDEN), full),
            pl.BlockSpec((1, HIDDEN), full),
            pl.BlockSpec((HIDDEN, HIDDEN), full),
            pl.BlockSpec((1, HIDDEN), full),
            pl.BlockSpec((USER_DIM, 256), full),
            pl.BlockSpec((1, 256), full),
            pl.BlockSpec((E, 256), full),
            pl.BlockSpec((E, 1), full),
            pl.BlockSpec((E + 1, HIDDEN), full),
            pl.BlockSpec((E + 1, 1), full),
        ],
        out_specs=[
            pl.BlockSpec((1, BLOCK_B), col),
            pl.BlockSpec((E, BLOCK_B), col),
        ],
        out_shape=[
            jax.ShapeDtypeStruct((1, B), jnp.float32),
            jax.ShapeDtypeStruct((E, B), jnp.float32),
        ],
        compiler_params=pltpu.CompilerParams(
            dimension_semantics=("parallel",),
        ),
    )(x, user_emb, enc_w1.astype(jnp.bfloat16), enc_b1[None, :],
      enc_w2.astype(jnp.bfloat16), enc_b2[None, :],
      gate_w1, gate_b1[None, :], gate_w2.T, gate_b2[:, None],
      w_out_t, b_out_t)

    return (total_t.reshape(B, 1), gate_t.T)


# speedup vs baseline: 1.0040x; 1.0040x over previous
import jax
import jax.numpy as jnp
from jax.experimental import pallas as pl
from jax.experimental.pallas import tpu as pltpu

B = 32768
INPUT_DIM = 768
HIDDEN = 768
USER_DIM = 128
E = 8
TOPK = 2

BLOCK_B = 1024


def _leaky_relu(v):
    return jnp.where(v >= 0, v, 0.2 * v)


def _dot(a, b):
    return jnp.dot(a, b, preferred_element_type=jnp.float32)


def _dot_t(a, b):
    return jax.lax.dot_general(
        a, b, ((( 1,), (1,)), ((), ())),
        preferred_element_type=jnp.float32)


def _fused_kernel(x_ref, u_ref, w1_ref, b1_ref, w2_ref, b2_ref,
                  gw1_ref, gb1_ref, gw2t_ref, gb2t_ref,
                  woutt_ref, boutt_ref,
                  total_ref, gate_ref):
    h = _leaky_relu(_dot(x_ref[...], w1_ref[...]) + b1_ref[...])
    features = _leaky_relu(_dot(h, w2_ref[...]) + b2_ref[...])

    g = jnp.maximum(_dot(u_ref[...], gw1_ref[...]) + gb1_ref[...], 0.0)
    logits = _dot_t(gw2t_ref[...], g) + gb2t_ref[...]

    m = jnp.max(logits, axis=0, keepdims=True)
    ex = jnp.exp(logits - m)
    gw = ex / jnp.sum(ex, axis=0, keepdims=True)

    lane = jax.lax.broadcasted_iota(jnp.int32, gw.shape, 0)
    m1 = jnp.max(gw, axis=0, keepdims=True)
    i1 = jnp.min(jnp.where(gw == m1, lane, E), axis=0, keepdims=True)
    sel1 = lane == i1
    gw_rest = jnp.where(sel1, -jnp.inf, gw)
    m2 = jnp.max(gw_rest, axis=0, keepdims=True)
    i2 = jnp.min(jnp.where(gw_rest == m2, lane, E), axis=0, keepdims=True)
    gw_masked = jnp.where(sel1 | (lane == i2), gw, 0.0)
    gw_final = gw_masked / (jnp.sum(gw_masked, axis=0, keepdims=True) + 1e-9)

    out9 = _dot_t(woutt_ref[...], features) + boutt_ref[...]
    total = jnp.sum(out9[:E] * gw_final, axis=0, keepdims=True) + out9[E:E + 1]

    total_ref[...] = total
    gate_ref[...] = gw_final


@jax.jit
def kernel(x, user_emb, enc_w1, enc_b1, enc_w2, enc_b2,
           gate_w1, gate_b1, gate_w2, gate_b2,
           expert_w, expert_b, shared_w, shared_b):
    w_out_t = jnp.concatenate([expert_w[:, :, 0], shared_w.T], axis=0)
    b_out_t = jnp.concatenate([expert_b[:, 0], shared_b])[:, None]

    grid = (B // BLOCK_B,)
    full = lambda i: (0, 0)
    row = lambda i: (i, 0)
    col = lambda i: (0, i)

    total_t, gate_t = pl.pallas_call(
        _fused_kernel,
        grid=grid,
        in_specs=[
            pl.BlockSpec((BLOCK_B, INPUT_DIM), row),
            pl.BlockSpec((BLOCK_B, USER_DIM), row),
            pl.BlockSpec((INPUT_DIM, HIDDEN), full),
            pl.BlockSpec((1, HIDDEN), full),
            pl.BlockSpec((HIDDEN, HIDDEN), full),
            pl.BlockSpec((1, HIDDEN), full),
            pl.BlockSpec((USER_DIM, 256), full),
            pl.BlockSpec((1, 256), full),
            pl.BlockSpec((E, 256), full),
            pl.BlockSpec((E, 1), full),
            pl.BlockSpec((E + 1, HIDDEN), full),
            pl.BlockSpec((E + 1, 1), full),
        ],
        out_specs=[
            pl.BlockSpec((1, BLOCK_B), col),
            pl.BlockSpec((E, BLOCK_B), col),
        ],
        out_shape=[
            jax.ShapeDtypeStruct((1, B), jnp.float32),
            jax.ShapeDtypeStruct((E, B), jnp.float32),
        ],
        compiler_params=pltpu.CompilerParams(
            dimension_semantics=("parallel",),
        ),
    )(x, user_emb, enc_w1, enc_b1[None, :], enc_w2, enc_b2[None, :],
      gate_w1, gate_b1[None, :], gate_w2.T, gate_b2[:, None],
      w_out_t, b_out_t)

    return (total_t.reshape(B, 1), gate_t.T)


# speedup vs baseline: 1.0830x; 1.0787x over previous
import jax
import jax.numpy as jnp
from jax.experimental import pallas as pl
from jax.experimental.pallas import tpu as pltpu

B = 32768
INPUT_DIM = 768
HIDDEN = 768
USER_DIM = 128
E = 8
TOPK = 2

BLOCK_B = 4096


def _leaky_relu(v):
    return jnp.maximum(v, 0.2 * v)


def _dot(a, b):
    return jnp.dot(a, b, preferred_element_type=jnp.float32)


def _dot_t(a, b):
    return jax.lax.dot_general(
        a, b, ((( 1,), (1,)), ((), ())),
        preferred_element_type=jnp.float32)


def _fused_kernel(x_ref, u_ref, w1_ref, b1_ref, w2_ref, b2_ref,
                  gw1_ref, gb1_ref, gw2t_ref, gb2t_ref,
                  woutt_ref, boutt_ref,
                  total_ref, gate_ref):
    h = _leaky_relu(_dot(x_ref[...], w1_ref[...]) + b1_ref[...])
    features = _leaky_relu(_dot(h, w2_ref[...]) + b2_ref[...])

    g = jnp.maximum(_dot(u_ref[...], gw1_ref[...]) + gb1_ref[...], 0.0)
    logits = _dot_t(gw2t_ref[...], g) + gb2t_ref[...]

    m = jnp.max(logits, axis=0, keepdims=True)
    ex = jnp.exp(logits - m)
    gw = ex / jnp.sum(ex, axis=0, keepdims=True)

    lane = jax.lax.broadcasted_iota(jnp.int32, gw.shape, 0)
    m1 = jnp.max(gw, axis=0, keepdims=True)
    i1 = jnp.min(jnp.where(gw == m1, lane, E), axis=0, keepdims=True)
    sel1 = lane == i1
    gw_rest = jnp.where(sel1, -jnp.inf, gw)
    m2 = jnp.max(gw_rest, axis=0, keepdims=True)
    i2 = jnp.min(jnp.where(gw_rest == m2, lane, E), axis=0, keepdims=True)
    gw_masked = jnp.where(sel1 | (lane == i2), gw, 0.0)
    gw_final = gw_masked / (jnp.sum(gw_masked, axis=0, keepdims=True) + 1e-9)

    out9 = _dot_t(woutt_ref[...], features) + boutt_ref[...]
    total = jnp.sum(out9[:E] * gw_final, axis=0, keepdims=True) + out9[E:E + 1]

    total_ref[...] = total
    gate_ref[...] = gw_final


@jax.jit
def kernel(x, user_emb, enc_w1, enc_b1, enc_w2, enc_b2,
           gate_w1, gate_b1, gate_w2, gate_b2,
           expert_w, expert_b, shared_w, shared_b):
    w_out_t = jnp.concatenate([expert_w[:, :, 0], shared_w.T], axis=0)
    b_out_t = jnp.concatenate([expert_b[:, 0], shared_b])[:, None]

    grid = (B // BLOCK_B,)
    full = lambda i: (0, 0)
    row = lambda i: (i, 0)
    col = lambda i: (0, i)

    total_t, gate_t = pl.pallas_call(
        _fused_kernel,
        grid=grid,
        in_specs=[
            pl.BlockSpec((BLOCK_B, INPUT_DIM), row),
            pl.BlockSpec((BLOCK_B, USER_DIM), row),
            pl.BlockSpec((INPUT_DIM, HIDDEN), full),
            pl.BlockSpec((1, HIDDEN), full),
            pl.BlockSpec((HIDDEN, HIDDEN), full),
            pl.BlockSpec((1, HIDDEN), full),
            pl.BlockSpec((USER_DIM, 256), full),
            pl.BlockSpec((1, 256), full),
            pl.BlockSpec((E, 256), full),
            pl.BlockSpec((E, 1), full),
            pl.BlockSpec((E + 1, HIDDEN), full),
            pl.BlockSpec((E + 1, 1), full),
        ],
        out_specs=[
            pl.BlockSpec((1, BLOCK_B), col),
            pl.BlockSpec((E, BLOCK_B), col),
        ],
        out_shape=[
            jax.ShapeDtypeStruct((1, B), jnp.float32),
            jax.ShapeDtypeStruct((E, B), jnp.float32),
        ],
        compiler_params=pltpu.CompilerParams(
            dimension_semantics=("parallel",),
        ),
    )(x, user_emb, enc_w1, enc_b1[None, :], enc_w2, enc_b2[None, :],
      gate_w1, gate_b1[None, :], gate_w2.T, gate_b2[:, None],
      w_out_t, b_out_t)

    return (total_t.reshape(B, 1), gate_t.T)
